# baseline (speedup 1.0000x reference)
import functools

import jax
import jax.numpy as jnp
from jax.experimental import pallas as pl
from jax.experimental.pallas import tpu as pltpu

POI_LEN = 38333
P = POI_LEN - 1
GF = 64
HALF = GF // 2
POI_DIM = 32
C = 64
OUT_DIM = 128


def _leaky(x, slope=0.01):
    return jnp.where(x >= 0, x, slope * x)


def _gcn_conv(x, row, col, W, b, n):
    h = x @ W
    loop = jnp.arange(n, dtype=row.dtype)
    r = jnp.concatenate([row, loop])
    c = jnp.concatenate([col, loop])
    deg = jnp.zeros((n,), x.dtype).at[c].add(1.0)
    dinv = jnp.where(deg > 0, 1.0 / jnp.sqrt(deg), 0.0)
    norm = dinv[r] * dinv[c]
    msg = h[r] * norm[:, None]
    out = jnp.zeros((n, h.shape[1]), x.dtype).at[c].add(msg)
    return out + b


def _gat_conv(x, row, col, W, a_src, a_dst, b, n, neg_slope=0.2):
    h = x @ W
    loop = jnp.arange(n, dtype=row.dtype)
    r = jnp.concatenate([row, loop])
    c = jnp.concatenate([col, loop])
    asrc = h @ a_src
    adst = h @ a_dst
    alpha = asrc[r] + adst[c]
    alpha = jnp.where(alpha >= 0, alpha, neg_slope * alpha)
    seg_max = jax.ops.segment_max(alpha, c, num_segments=n)
    alpha = jnp.exp(alpha - seg_max[c])
    seg_sum = jax.ops.segment_sum(alpha, c, num_segments=n)
    alpha = alpha / (seg_sum[c] + 1e-16)
    out = jnp.zeros((n, h.shape[1]), x.dtype).at[c].add(h[r] * alpha[:, None])
    return out + b


def _graph_norm(x, w, b, ms, eps=1e-5):
    mean = x.mean(axis=0, keepdims=True)
    out = x - ms * mean
    var = (out * out).mean(axis=0, keepdims=True)
    return w * out / jnp.sqrt(var + eps) + b


def _gcn_unit(x, row, col, p, n):
    t = _gcn_conv(x, row, col, p['gcn_W'], p['gcn_b'], n)
    t = _graph_norm(t, p['gn_w'], p['gn_b'], p['gn_ms'])
    t = _leaky(t)
    x = x + t
    t = _gat_conv(x, row, col, p['gat_W'], p['gat_asrc'], p['gat_adst'], p['gat_b'], n)
    t = _graph_norm(t, p['gn_w'], p['gn_b'], p['gn_ms'])
    t = _leaky(t)
    return x + t


_PB = 4096


def _head_body(feat_ref, w1_ref, acc_ref):
    k = pl.program_id(0)
    @pl.when(k == 0)
    def _init():
        acc_ref[...] = jnp.zeros_like(acc_ref)
    acc_ref[...] += jnp.dot(feat_ref[...], w1_ref[...],
                            preferred_element_type=jnp.float32)


def _head_finish_body(acc_ref, b1_ref, w2_ref, b2_ref, out_ref):
    h = _leaky(acc_ref[...] + b1_ref[...])
    out_ref[...] = jnp.dot(h, w2_ref[...],
                           preferred_element_type=jnp.float32) + b2_ref[...]


@jax.jit
def _mlp_head(feat, w1, b1, w2, b2):
    n = feat.shape[0]
    npad = ((n + _PB - 1) // _PB) * _PB
    featp = jnp.pad(feat, (0, npad - n)).reshape(1, npad)
    w1p = jnp.pad(w1, ((0, npad - n), (0, 0)))
    nk = npad // _PB
    acc = pl.pallas_call(
        _head_body,
        grid=(nk,),
        in_specs=[
            pl.BlockSpec((1, _PB), lambda k: (0, k)),
            pl.BlockSpec((_PB, 128), lambda k: (k, 0)),
        ],
        out_specs=pl.BlockSpec((1, 128), lambda k: (0, 0)),
        out_shape=jax.ShapeDtypeStruct((1, 128), jnp.float32),
    )(featp, w1p)
    out = pl.pallas_call(
        _head_finish_body,
        out_shape=jax.ShapeDtypeStruct((1, OUT_DIM), jnp.float32),
    )(acc, b1.reshape(1, 128), w2, b2.reshape(1, OUT_DIM))
    return out.reshape(OUT_DIM)


def kernel(x, edge_index, mask, weight, params):
    row, col = edge_index[0], edge_index[1]
    poi = jnp.where(mask, x, 0)[:, :HALF].reshape(P, -1).astype(jnp.int32)
    dist = jnp.where(mask, 0.0, x)[:, HALF:].reshape(P, -1)
    emb_poi = params['emb'][poi]
    feat = jnp.concatenate([emb_poi.reshape(P, -1), dist], axis=1)
    feat = _leaky(_gcn_conv(feat, row, col, params['cov_in_W'], params['cov_in_b'], P))
    for p in params['layers']:
        feat = _gcn_unit(feat, row, col, p, P)
    feat = _leaky(_gcn_conv(feat, row, col, params['cov_out_W'], params['cov_out_b'], P))
    feat = feat.reshape(-1)
    return _mlp_head(feat, params['fc1_W'], params['fc1_b'],
                     params['fc2_W'], params['fc2_b'])


# speedup vs baseline: 1.4636x; 1.4636x over previous
import functools

import jax
import jax.numpy as jnp
from jax import lax
from jax.experimental import pallas as pl
from jax.experimental.pallas import tpu as pltpu
from jax.experimental.pallas import tpu_sc as plsc

POI_LEN = 38333
P = POI_LEN - 1
GF = 64
HALF = GF // 2
POI_DIM = 32
C = 64
OUT_DIM = 128
E = P * 16

PPAD = 38400
NSUB = 16
CHUNK = 128
EPAD = 614400
NCH = EPAD // (NSUB * CHUNK)
ROWS_PER = PPAD // NSUB


def _leaky(x, slope=0.01):
    return jnp.where(x >= 0, x, slope * x)


KSTAGE = 10


def _agg_body(hdcat, ridx0, ridx1, cidx, out, ridx_v, cidx_v, rows_v,
              shared, sem):
    ci = lax.axis_index("c")
    si = lax.axis_index("s")

    base = ci * PPAD + si * ROWS_PER
    pltpu.sync_copy(hdcat.at[pl.ds(base, ROWS_PER)],
                    shared.at[pl.ds(si * ROWS_PER, ROWS_PER)])
    plsc.subcore_barrier()

    def outer(o, carry):
        @pl.when(ci == 0)
        def _():
            pltpu.sync_copy(ridx0.at[si, pl.ds(o * KSTAGE, KSTAGE)], ridx_v)

        @pl.when(ci == 1)
        def _():
            pltpu.sync_copy(ridx1.at[si, pl.ds(o * KSTAGE, KSTAGE)], ridx_v)

        pltpu.sync_copy(cidx.at[si, pl.ds(o * KSTAGE, KSTAGE)], cidx_v)

        def step(g, c2):
            pltpu.async_copy(hdcat.at[ridx_v.at[g]], rows_v, sem).wait()
            pltpu.sync_copy(rows_v, shared.at[cidx_v.at[g]], add=True)
            return c2

        lax.fori_loop(0, KSTAGE, step, 0)
        return carry

    lax.fori_loop(0, NCH // KSTAGE, outer, 0)
    plsc.subcore_barrier()
    pltpu.sync_copy(shared.at[pl.ds(si * ROWS_PER, ROWS_PER)],
                    out.at[pl.ds(base, ROWS_PER)])


_agg_call = functools.partial(
    pl.kernel,
    out_type=jax.ShapeDtypeStruct((2 * PPAD, 32), jnp.float32),
    mesh=plsc.VectorSubcoreMesh(core_axis_name="c", subcore_axis_name="s"),
    scratch_types=[
        pltpu.VMEM((KSTAGE, CHUNK), jnp.int32),
        pltpu.VMEM((KSTAGE, CHUNK), jnp.int32),
        pltpu.VMEM((CHUNK, 32), jnp.float32),
        pltpu.VMEM_SHARED((PPAD, 32), jnp.float32),
        pltpu.SemaphoreType.DMA,
    ],
    compiler_params=pltpu.CompilerParams(use_tc_tiling_on_sc=False),
)(_agg_body)


def _prep_edges(row, col):
    npad = EPAD - E
    r = jnp.concatenate([row, jnp.zeros((npad,), row.dtype)])
    c = jnp.concatenate([col, jnp.full((npad,), P, col.dtype)])
    r = r.reshape(NSUB, NCH, CHUNK).astype(jnp.int32)
    c = c.reshape(NSUB, NCH, CHUNK).astype(jnp.int32)
    return r, r + PPAD, c


def _sc_agg(hd, ridx0, ridx1, cidx):
    hdp = jnp.pad(hd, ((0, PPAD - P), (0, 0)))
    hdcat = jnp.concatenate([hdp[:, :32], hdp[:, 32:]], axis=0)
    acc = _agg_call(hdcat, ridx0, ridx1, cidx)
    return jnp.concatenate([acc[:P], acc[PPAD:PPAD + P]], axis=1)


def _gcn_conv(x, eidx, W, b, dinv):
    hd = (x @ W) * dinv[:, None]
    acc = _sc_agg(hd, *eidx)
    return dinv[:, None] * acc + b


def _gcn_conv_out(x, eidx, W, b, dinv):
    acc = _sc_agg(x * dinv[:, None], *eidx)
    return dinv[:, None] * (acc @ W) + b


def _gat_conv(x, row, col, W, a_src, a_dst, b, n, neg_slope=0.2):
    h = x @ W
    loop = jnp.arange(n, dtype=row.dtype)
    r = jnp.concatenate([row, loop])
    c = jnp.concatenate([col, loop])
    asrc = h @ a_src
    adst = h @ a_dst
    alpha = asrc[r] + adst[c]
    alpha = jnp.where(alpha >= 0, alpha, neg_slope * alpha)
    seg_max = jax.ops.segment_max(alpha, c, num_segments=n)
    alpha = jnp.exp(alpha - seg_max[c])
    seg_sum = jax.ops.segment_sum(alpha, c, num_segments=n)
    alpha = alpha / (seg_sum[c] + 1e-16)
    out = jnp.zeros((n, h.shape[1]), x.dtype).at[c].add(h[r] * alpha[:, None])
    return out + b


def _graph_norm(x, w, b, ms, eps=1e-5):
    mean = x.mean(axis=0, keepdims=True)
    out = x - ms * mean
    var = (out * out).mean(axis=0, keepdims=True)
    return w * out / jnp.sqrt(var + eps) + b


def _gcn_unit(x, row, col, eidx, p, dinv):
    t = _gcn_conv(x, eidx, p['gcn_W'], p['gcn_b'], dinv)
    t = _graph_norm(t, p['gn_w'], p['gn_b'], p['gn_ms'])
    t = _leaky(t)
    x = x + t
    t = _gat_conv(x, row, col, p['gat_W'], p['gat_asrc'], p['gat_adst'],
                  p['gat_b'], P)
    t = _graph_norm(t, p['gn_w'], p['gn_b'], p['gn_ms'])
    t = _leaky(t)
    return x + t


_PB = 4096


def _head_body(feat_ref, w1_ref, acc_ref):
    k = pl.program_id(0)

    @pl.when(k == 0)
    def _init():
        acc_ref[...] = jnp.zeros_like(acc_ref)

    acc_ref[...] += jnp.dot(feat_ref[...], w1_ref[...],
                            preferred_element_type=jnp.float32)


def _head_finish_body(acc_ref, b1_ref, w2_ref, b2_ref, out_ref):
    h = _leaky(acc_ref[...] + b1_ref[...])
    out_ref[...] = jnp.dot(h, w2_ref[...],
                           preferred_element_type=jnp.float32) + b2_ref[...]


def _mlp_head(feat, w1, b1, w2, b2):
    n = feat.shape[0]
    npad = ((n + _PB - 1) // _PB) * _PB
    featp = jnp.pad(feat, (0, npad - n)).reshape(1, npad)
    w1p = jnp.pad(w1, ((0, npad - n), (0, 0)))
    nk = npad // _PB
    acc = pl.pallas_call(
        _head_body,
        grid=(nk,),
        in_specs=[
            pl.BlockSpec((1, _PB), lambda k: (0, k)),
            pl.BlockSpec((_PB, 128), lambda k: (k, 0)),
        ],
        out_specs=pl.BlockSpec((1, 128), lambda k: (0, 0)),
        out_shape=jax.ShapeDtypeStruct((1, 128), jnp.float32),
    )(featp, w1p)
    out = pl.pallas_call(
        _head_finish_body,
        out_shape=jax.ShapeDtypeStruct((1, OUT_DIM), jnp.float32),
    )(acc, b1.reshape(1, 128), w2, b2.reshape(1, OUT_DIM))
    return out.reshape(OUT_DIM)


def kernel(x, edge_index, mask, weight, params):
    row, col = edge_index[0], edge_index[1]
    poi = jnp.where(mask, x, 0)[:, :HALF].reshape(P, -1).astype(jnp.int32)
    dist = jnp.where(mask, 0.0, x)[:, HALF:].reshape(P, -1)
    emb_poi = params['emb'][poi]
    feat = jnp.concatenate([emb_poi.reshape(P, -1), dist], axis=1)

    deg = jnp.ones((P,), jnp.float32).at[col].add(1.0)
    dinv = 1.0 / jnp.sqrt(deg)
    eidx = _prep_edges(row, col)

    feat = _leaky(_gcn_conv(feat, eidx, params['cov_in_W'],
                            params['cov_in_b'], dinv))
    for p in params['layers']:
        feat = _gcn_unit(feat, row, col, eidx, p, dinv)
    feat = _leaky(_gcn_conv_out(feat, eidx, params['cov_out_W'],
                                params['cov_out_b'], dinv))
    feat = feat.reshape(-1)
    return _mlp_head(feat, params['fc1_W'], params['fc1_b'],
                     params['fc2_W'], params['fc2_b'])


# speedup vs baseline: 1.8123x; 1.2382x over previous
import functools

import jax
import jax.numpy as jnp
from jax import lax
from jax.experimental import pallas as pl
from jax.experimental.pallas import tpu as pltpu
from jax.experimental.pallas import tpu_sc as plsc

POI_LEN = 38333
P = POI_LEN - 1
GF = 64
HALF = GF // 2
POI_DIM = 32
C = 64
OUT_DIM = 128
E = P * 16

PPAD = 38400
NSUB = 16
CHUNK = 128
EPAD = 614400
NCH = EPAD // (NSUB * CHUNK)
ROWS_PER = PPAD // NSUB


def _leaky(x, slope=0.01):
    return jnp.where(x >= 0, x, slope * x)


KSTAGE = 10


def _agg_body(hdcat, ridx0, ridx1, cidx, out, ridx_v, cidx_v, rows_v,
              shared, sem):
    ci = lax.axis_index("c")
    si = lax.axis_index("s")

    base = ci * PPAD + si * ROWS_PER
    pltpu.sync_copy(hdcat.at[pl.ds(base, ROWS_PER)],
                    shared.at[pl.ds(si * ROWS_PER, ROWS_PER)])
    plsc.subcore_barrier()

    def outer(o, carry):
        @pl.when(ci == 0)
        def _():
            pltpu.sync_copy(ridx0.at[si, pl.ds(o * KSTAGE, KSTAGE)], ridx_v)

        @pl.when(ci == 1)
        def _():
            pltpu.sync_copy(ridx1.at[si, pl.ds(o * KSTAGE, KSTAGE)], ridx_v)

        pltpu.sync_copy(cidx.at[si, pl.ds(o * KSTAGE, KSTAGE)], cidx_v)

        def step(g, c2):
            pltpu.async_copy(hdcat.at[ridx_v.at[g]], rows_v, sem).wait()
            pltpu.sync_copy(rows_v, shared.at[cidx_v.at[g]], add=True)
            return c2

        lax.fori_loop(0, KSTAGE, step, 0)
        return carry

    lax.fori_loop(0, NCH // KSTAGE, outer, 0)
    plsc.subcore_barrier()
    pltpu.sync_copy(shared.at[pl.ds(si * ROWS_PER, ROWS_PER)],
                    out.at[pl.ds(base, ROWS_PER)])


_GATHER_DN = lax.GatherDimensionNumbers(
    offset_dims=(), collapsed_slice_dims=(0,), start_index_map=(0,))


def _lane_bcast(v16, l):
    idx = jnp.full((16, 1), l, jnp.int32)
    return lax.gather(v16, idx, _GATHER_DN, slice_sizes=(1,),
                      mode=lax.GatherScatterMode.PROMISE_IN_BOUNDS)


def _gat_body(hcat, initcat, ridx0, ridx1, cidx, wgt, out, ridx_v, cidx_v,
              w_v, rows_v, shared, sem):
    ci = lax.axis_index("c")
    si = lax.axis_index("s")

    base = ci * PPAD + si * ROWS_PER
    pltpu.sync_copy(initcat.at[pl.ds(base, ROWS_PER)],
                    shared.at[pl.ds(si * ROWS_PER, ROWS_PER)])
    plsc.subcore_barrier()

    def outer(o, carry):
        @pl.when(ci == 0)
        def _():
            pltpu.sync_copy(ridx0.at[si, pl.ds(o * KSTAGE, KSTAGE)], ridx_v)

        @pl.when(ci == 1)
        def _():
            pltpu.sync_copy(ridx1.at[si, pl.ds(o * KSTAGE, KSTAGE)], ridx_v)

        pltpu.sync_copy(cidx.at[si, pl.ds(o * KSTAGE, KSTAGE)], cidx_v)
        pltpu.sync_copy(wgt.at[si, pl.ds(o * KSTAGE, KSTAGE)], w_v)

        def step(g, c2):
            pltpu.async_copy(hcat.at[ridx_v.at[g]], rows_v, sem).wait()
            for j in range(CHUNK // 16):
                w16 = w_v[g, j * 16:(j + 1) * 16]
                for l in range(16):
                    e = j * 16 + l
                    b = _lane_bcast(w16, l)
                    rows_v[e, 0:16] = rows_v[e, 0:16] * b
                    rows_v[e, 16:32] = rows_v[e, 16:32] * b
            pltpu.sync_copy(rows_v, shared.at[cidx_v.at[g]], add=True)
            return c2

        lax.fori_loop(0, KSTAGE, step, 0)
        return carry

    lax.fori_loop(0, NCH // KSTAGE, outer, 0)
    plsc.subcore_barrier()
    pltpu.sync_copy(shared.at[pl.ds(si * ROWS_PER, ROWS_PER)],
                    out.at[pl.ds(base, ROWS_PER)])


_gat_call = functools.partial(
    pl.kernel,
    out_type=jax.ShapeDtypeStruct((2 * PPAD, 32), jnp.float32),
    mesh=plsc.VectorSubcoreMesh(core_axis_name="c", subcore_axis_name="s"),
    scratch_types=[
        pltpu.VMEM((KSTAGE, CHUNK), jnp.int32),
        pltpu.VMEM((KSTAGE, CHUNK), jnp.int32),
        pltpu.VMEM((KSTAGE, CHUNK), jnp.float32),
        pltpu.VMEM((CHUNK, 32), jnp.float32),
        pltpu.VMEM_SHARED((PPAD, 32), jnp.float32),
        pltpu.SemaphoreType.DMA,
    ],
    compiler_params=pltpu.CompilerParams(use_tc_tiling_on_sc=False),
)(_gat_body)


_agg_call = functools.partial(
    pl.kernel,
    out_type=jax.ShapeDtypeStruct((2 * PPAD, 32), jnp.float32),
    mesh=plsc.VectorSubcoreMesh(core_axis_name="c", subcore_axis_name="s"),
    scratch_types=[
        pltpu.VMEM((KSTAGE, CHUNK), jnp.int32),
        pltpu.VMEM((KSTAGE, CHUNK), jnp.int32),
        pltpu.VMEM((CHUNK, 32), jnp.float32),
        pltpu.VMEM_SHARED((PPAD, 32), jnp.float32),
        pltpu.SemaphoreType.DMA,
    ],
    compiler_params=pltpu.CompilerParams(use_tc_tiling_on_sc=False),
)(_agg_body)


def _prep_edges(row, col):
    npad = EPAD - E
    r = jnp.concatenate([row, jnp.zeros((npad,), row.dtype)])
    c = jnp.concatenate([col, jnp.full((npad,), P, col.dtype)])
    r = r.reshape(NSUB, NCH, CHUNK).astype(jnp.int32)
    c = c.reshape(NSUB, NCH, CHUNK).astype(jnp.int32)
    return r, r + PPAD, c


def _sc_agg(hd, ridx0, ridx1, cidx):
    hdp = jnp.pad(hd, ((0, PPAD - P), (0, 0)))
    hdcat = jnp.concatenate([hdp[:, :32], hdp[:, 32:]], axis=0)
    acc = _agg_call(hdcat, ridx0, ridx1, cidx)
    return jnp.concatenate([acc[:P], acc[PPAD:PPAD + P]], axis=1)


def _gcn_conv(x, eidx, W, b, dinv):
    hd = (x @ W) * dinv[:, None]
    acc = _sc_agg(hd, *eidx)
    return dinv[:, None] * acc + b


def _gcn_conv_out(x, eidx, W, b, dinv):
    acc = _sc_agg(x * dinv[:, None], *eidx)
    return dinv[:, None] * (acc @ W) + b


def _sc_gat_agg(h, init, w_edge, eidx):
    ridx0, ridx1, cidx = eidx
    hp = jnp.pad(h, ((0, PPAD - P), (0, 0)))
    hcat = jnp.concatenate([hp[:, :32], hp[:, 32:]], axis=0)
    ip = jnp.pad(init, ((0, PPAD - P), (0, 0)))
    icat = jnp.concatenate([ip[:, :32], ip[:, 32:]], axis=0)
    wp = jnp.concatenate([w_edge, jnp.zeros((EPAD - E,), jnp.float32)])
    wp = wp.reshape(NSUB, NCH, CHUNK)
    num = _gat_call(hcat, icat, ridx0, ridx1, cidx, wp)
    return jnp.concatenate([num[:P], num[PPAD:PPAD + P]], axis=1)


def _gat_conv(x, row, col, eidx, W, a_src, a_dst, b, neg_slope=0.2):
    h = x @ W
    asrc = h @ a_src
    adst = h @ a_dst
    a_edge = asrc[row] + adst[col]
    a_edge = jnp.where(a_edge >= 0, a_edge, neg_slope * a_edge)
    a_self = asrc + adst
    a_self = jnp.where(a_self >= 0, a_self, neg_slope * a_self)
    seg_max = jax.ops.segment_max(a_edge, col, num_segments=P)
    seg_max = jnp.maximum(seg_max, a_self)
    w_edge = jnp.exp(a_edge - seg_max[col])
    w_self = jnp.exp(a_self - seg_max)
    seg_sum = jax.ops.segment_sum(w_edge, col, num_segments=P) + w_self
    num = _sc_gat_agg(h, h * w_self[:, None], w_edge, eidx)
    return num / (seg_sum + 1e-16)[:, None] + b


def _graph_norm(x, w, b, ms, eps=1e-5):
    mean = x.mean(axis=0, keepdims=True)
    out = x - ms * mean
    var = (out * out).mean(axis=0, keepdims=True)
    return w * out / jnp.sqrt(var + eps) + b


def _gcn_unit(x, row, col, eidx, p, dinv):
    t = _gcn_conv(x, eidx, p['gcn_W'], p['gcn_b'], dinv)
    t = _graph_norm(t, p['gn_w'], p['gn_b'], p['gn_ms'])
    t = _leaky(t)
    x = x + t
    t = _gat_conv(x, row, col, eidx, p['gat_W'], p['gat_asrc'],
                  p['gat_adst'], p['gat_b'])
    t = _graph_norm(t, p['gn_w'], p['gn_b'], p['gn_ms'])
    t = _leaky(t)
    return x + t


_PB = 4096


def _head_body(feat_ref, w1_ref, acc_ref):
    k = pl.program_id(0)

    @pl.when(k == 0)
    def _init():
        acc_ref[...] = jnp.zeros_like(acc_ref)

    acc_ref[...] += jnp.dot(feat_ref[...], w1_ref[...],
                            preferred_element_type=jnp.float32)


def _head_finish_body(acc_ref, b1_ref, w2_ref, b2_ref, out_ref):
    h = _leaky(acc_ref[...] + b1_ref[...])
    out_ref[...] = jnp.dot(h, w2_ref[...],
                           preferred_element_type=jnp.float32) + b2_ref[...]


def _mlp_head(feat, w1, b1, w2, b2):
    n = feat.shape[0]
    npad = ((n + _PB - 1) // _PB) * _PB
    featp = jnp.pad(feat, (0, npad - n)).reshape(1, npad)
    w1p = jnp.pad(w1, ((0, npad - n), (0, 0)))
    nk = npad // _PB
    acc = pl.pallas_call(
        _head_body,
        grid=(nk,),
        in_specs=[
            pl.BlockSpec((1, _PB), lambda k: (0, k)),
            pl.BlockSpec((_PB, 128), lambda k: (k, 0)),
        ],
        out_specs=pl.BlockSpec((1, 128), lambda k: (0, 0)),
        out_shape=jax.ShapeDtypeStruct((1, 128), jnp.float32),
    )(featp, w1p)
    out = pl.pallas_call(
        _head_finish_body,
        out_shape=jax.ShapeDtypeStruct((1, OUT_DIM), jnp.float32),
    )(acc, b1.reshape(1, 128), w2, b2.reshape(1, OUT_DIM))
    return out.reshape(OUT_DIM)


def kernel(x, edge_index, mask, weight, params):
    row, col = edge_index[0], edge_index[1]
    poi = jnp.where(mask, x, 0)[:, :HALF].reshape(P, -1).astype(jnp.int32)
    dist = jnp.where(mask, 0.0, x)[:, HALF:].reshape(P, -1)
    emb_poi = params['emb'][poi]
    feat = jnp.concatenate([emb_poi.reshape(P, -1), dist], axis=1)

    deg = jnp.ones((P,), jnp.float32).at[col].add(1.0)
    dinv = 1.0 / jnp.sqrt(deg)
    eidx = _prep_edges(row, col)

    feat = _leaky(_gcn_conv(feat, eidx, params['cov_in_W'],
                            params['cov_in_b'], dinv))
    for p in params['layers']:
        feat = _gcn_unit(feat, row, col, eidx, p, dinv)
    feat = _leaky(_gcn_conv_out(feat, eidx, params['cov_out_W'],
                                params['cov_out_b'], dinv))
    feat = feat.reshape(-1)
    return _mlp_head(feat, params['fc1_W'], params['fc1_b'],
                     params['fc2_W'], params['fc2_b'])


# speedup vs baseline: 1.8605x; 1.0266x over previous
import functools

import jax
import jax.numpy as jnp
from jax import lax
from jax.experimental import pallas as pl
from jax.experimental.pallas import tpu as pltpu
from jax.experimental.pallas import tpu_sc as plsc

POI_LEN = 38333
P = POI_LEN - 1
GF = 64
HALF = GF // 2
POI_DIM = 32
C = 64
OUT_DIM = 128
E = P * 16

PPAD = 38400
NSUB = 16
CHUNK = 128
EPAD = 614400
NCH = EPAD // (NSUB * CHUNK)
ROWS_PER = PPAD // NSUB


def _leaky(x, slope=0.01):
    return jnp.where(x >= 0, x, slope * x)


KSTAGE = 10


def _agg_body(hdcat, ridx0, ridx1, cidx, out, ridx_v, cidx_v, rows_v,
              shared, sem):
    ci = lax.axis_index("c")
    si = lax.axis_index("s")

    base = ci * PPAD + si * ROWS_PER
    pltpu.sync_copy(hdcat.at[pl.ds(base, ROWS_PER)],
                    shared.at[pl.ds(si * ROWS_PER, ROWS_PER)])
    plsc.subcore_barrier()

    def outer(o, carry):
        @pl.when(ci == 0)
        def _():
            pltpu.sync_copy(ridx0.at[si, pl.ds(o * KSTAGE, KSTAGE)], ridx_v)

        @pl.when(ci == 1)
        def _():
            pltpu.sync_copy(ridx1.at[si, pl.ds(o * KSTAGE, KSTAGE)], ridx_v)

        pltpu.sync_copy(cidx.at[si, pl.ds(o * KSTAGE, KSTAGE)], cidx_v)

        def step(g, c2):
            pltpu.async_copy(hdcat.at[ridx_v.at[g]], rows_v, sem).wait()
            pltpu.sync_copy(rows_v, shared.at[cidx_v.at[g]], add=True)
            return c2

        lax.fori_loop(0, KSTAGE, step, 0)
        return carry

    lax.fori_loop(0, NCH // KSTAGE, outer, 0)
    plsc.subcore_barrier()
    pltpu.sync_copy(shared.at[pl.ds(si * ROWS_PER, ROWS_PER)],
                    out.at[pl.ds(base, ROWS_PER)])


_GATHER_DN = lax.GatherDimensionNumbers(
    offset_dims=(), collapsed_slice_dims=(0,), start_index_map=(0,))


def _lane_bcast(v16, l):
    idx = jnp.full((16, 1), l, jnp.int32)
    return lax.gather(v16, idx, _GATHER_DN, slice_sizes=(1,),
                      mode=lax.GatherScatterMode.PROMISE_IN_BOUNDS)


def _gat_body(hcat, initcat, ridx0, ridx1, cidx, wgt, out, ridx_v, cidx_v,
              w_v, rows_v, shared, sem):
    ci = lax.axis_index("c")
    si = lax.axis_index("s")

    base = ci * PPAD + si * ROWS_PER
    pltpu.sync_copy(initcat.at[pl.ds(base, ROWS_PER)],
                    shared.at[pl.ds(si * ROWS_PER, ROWS_PER)])
    plsc.subcore_barrier()

    def outer(o, carry):
        @pl.when(ci == 0)
        def _():
            pltpu.sync_copy(ridx0.at[si, pl.ds(o * KSTAGE, KSTAGE)], ridx_v)

        @pl.when(ci == 1)
        def _():
            pltpu.sync_copy(ridx1.at[si, pl.ds(o * KSTAGE, KSTAGE)], ridx_v)

        pltpu.sync_copy(cidx.at[si, pl.ds(o * KSTAGE, KSTAGE)], cidx_v)
        pltpu.sync_copy(wgt.at[si, pl.ds(o * KSTAGE, KSTAGE)], w_v)

        def step(g, c2):
            pltpu.async_copy(hcat.at[ridx_v.at[g]], rows_v, sem).wait()
            for j in range(CHUNK // 16):
                w16 = w_v[g, j * 16:(j + 1) * 16]
                for l in range(16):
                    e = j * 16 + l
                    b = _lane_bcast(w16, l)
                    rows_v[e, 0:16] = rows_v[e, 0:16] * b
                    rows_v[e, 16:32] = rows_v[e, 16:32] * b
            pltpu.sync_copy(rows_v, shared.at[cidx_v.at[g]], add=True)
            return c2

        lax.fori_loop(0, KSTAGE, step, 0)
        return carry

    lax.fori_loop(0, NCH // KSTAGE, outer, 0)
    plsc.subcore_barrier()
    pltpu.sync_copy(shared.at[pl.ds(si * ROWS_PER, ROWS_PER)],
                    out.at[pl.ds(base, ROWS_PER)])


_gat_call = functools.partial(
    pl.kernel,
    out_type=jax.ShapeDtypeStruct((2 * PPAD, 32), jnp.float32),
    mesh=plsc.VectorSubcoreMesh(core_axis_name="c", subcore_axis_name="s"),
    scratch_types=[
        pltpu.VMEM((KSTAGE, CHUNK), jnp.int32),
        pltpu.VMEM((KSTAGE, CHUNK), jnp.int32),
        pltpu.VMEM((KSTAGE, CHUNK), jnp.float32),
        pltpu.VMEM((CHUNK, 32), jnp.float32),
        pltpu.VMEM_SHARED((PPAD, 32), jnp.float32),
        pltpu.SemaphoreType.DMA,
    ],
    compiler_params=pltpu.CompilerParams(use_tc_tiling_on_sc=False),
)(_gat_body)


_agg_call = functools.partial(
    pl.kernel,
    out_type=jax.ShapeDtypeStruct((2 * PPAD, 32), jnp.float32),
    mesh=plsc.VectorSubcoreMesh(core_axis_name="c", subcore_axis_name="s"),
    scratch_types=[
        pltpu.VMEM((KSTAGE, CHUNK), jnp.int32),
        pltpu.VMEM((KSTAGE, CHUNK), jnp.int32),
        pltpu.VMEM((CHUNK, 32), jnp.float32),
        pltpu.VMEM_SHARED((PPAD, 32), jnp.float32),
        pltpu.SemaphoreType.DMA,
    ],
    compiler_params=pltpu.CompilerParams(use_tc_tiling_on_sc=False),
)(_agg_body)


def _prep_edges(row, col):
    npad = EPAD - E
    r = jnp.concatenate([row, jnp.zeros((npad,), row.dtype)])
    c = jnp.concatenate([col, jnp.full((npad,), P, col.dtype)])
    r = r.reshape(NSUB, NCH, CHUNK).astype(jnp.int32)
    c = c.reshape(NSUB, NCH, CHUNK).astype(jnp.int32)
    return r, r + PPAD, c


def _sc_agg(hd, ridx0, ridx1, cidx):
    hdp = jnp.pad(hd, ((0, PPAD - P), (0, 0)))
    hdcat = jnp.concatenate([hdp[:, :32], hdp[:, 32:]], axis=0)
    acc = _agg_call(hdcat, ridx0, ridx1, cidx)
    return jnp.concatenate([acc[:P], acc[PPAD:PPAD + P]], axis=1)


IPAD = 1228800
ICH = IPAD // (32 * CHUNK)


def _emb_body(tab, idx, out, idx_v, rows_v, sem):
    ci = lax.axis_index("c")
    si = lax.axis_index("s")
    w = si * 2 + ci

    def outer(o, carry):
        pltpu.sync_copy(idx.at[w, pl.ds(o * KSTAGE, KSTAGE)], idx_v)

        def step(g, c2):
            pltpu.async_copy(tab.at[idx_v.at[g]], rows_v, sem).wait()
            base = w * (ICH * CHUNK) + (o * KSTAGE + g) * CHUNK
            pltpu.sync_copy(rows_v, out.at[pl.ds(base, CHUNK)])
            return c2

        lax.fori_loop(0, KSTAGE, step, 0)
        return carry

    lax.fori_loop(0, ICH // KSTAGE, outer, 0)


_emb_call = functools.partial(
    pl.kernel,
    out_type=jax.ShapeDtypeStruct((IPAD, 32), jnp.float32),
    mesh=plsc.VectorSubcoreMesh(core_axis_name="c", subcore_axis_name="s"),
    scratch_types=[
        pltpu.VMEM((KSTAGE, CHUNK), jnp.int32),
        pltpu.VMEM((CHUNK, 32), jnp.float32),
        pltpu.SemaphoreType.DMA,
    ],
    compiler_params=pltpu.CompilerParams(use_tc_tiling_on_sc=False),
)(_emb_body)


def _sc_emb_lookup(emb, poi):
    flat = poi.reshape(-1)
    flat = jnp.concatenate(
        [flat, jnp.zeros((IPAD - flat.shape[0],), flat.dtype)])
    rows = _emb_call(emb, flat.reshape(32, ICH, CHUNK))
    return rows[:P * HALF].reshape(P, -1)


def _gcn_conv(x, eidx, W, b, dinv):
    hd = (x @ W) * dinv[:, None]
    acc = _sc_agg(hd, *eidx)
    return dinv[:, None] * acc + b


def _gcn_conv_out(x, eidx, W, b, dinv):
    acc = _sc_agg(x * dinv[:, None], *eidx)
    return dinv[:, None] * (acc @ W) + b


def _sc_gat_agg(h, init, w_edge, eidx):
    ridx0, ridx1, cidx = eidx
    hp = jnp.pad(h, ((0, PPAD - P), (0, 0)))
    hcat = jnp.concatenate([hp[:, :32], hp[:, 32:]], axis=0)
    ip = jnp.pad(init, ((0, PPAD - P), (0, 0)))
    icat = jnp.concatenate([ip[:, :32], ip[:, 32:]], axis=0)
    wp = jnp.concatenate([w_edge, jnp.zeros((EPAD - E,), jnp.float32)])
    wp = wp.reshape(NSUB, NCH, CHUNK)
    num = _gat_call(hcat, icat, ridx0, ridx1, cidx, wp)
    return jnp.concatenate([num[:P], num[PPAD:PPAD + P]], axis=1)


def _gat_conv(x, row, col, eidx, W, a_src, a_dst, b, neg_slope=0.2):
    h = x @ W
    asrc = h @ a_src
    adst = h @ a_dst
    a_edge = asrc[row] + adst[col]
    a_edge = jnp.where(a_edge >= 0, a_edge, neg_slope * a_edge)
    a_self = asrc + adst
    a_self = jnp.where(a_self >= 0, a_self, neg_slope * a_self)
    seg_max = jax.ops.segment_max(a_edge, col, num_segments=P)
    seg_max = jnp.maximum(seg_max, a_self)
    w_edge = jnp.exp(a_edge - seg_max[col])
    w_self = jnp.exp(a_self - seg_max)
    seg_sum = jax.ops.segment_sum(w_edge, col, num_segments=P) + w_self
    num = _sc_gat_agg(h, h * w_self[:, None], w_edge, eidx)
    return num / (seg_sum + 1e-16)[:, None] + b


def _graph_norm(x, w, b, ms, eps=1e-5):
    mean = x.mean(axis=0, keepdims=True)
    out = x - ms * mean
    var = (out * out).mean(axis=0, keepdims=True)
    return w * out / jnp.sqrt(var + eps) + b


def _gcn_unit(x, row, col, eidx, p, dinv):
    t = _gcn_conv(x, eidx, p['gcn_W'], p['gcn_b'], dinv)
    t = _graph_norm(t, p['gn_w'], p['gn_b'], p['gn_ms'])
    t = _leaky(t)
    x = x + t
    t = _gat_conv(x, row, col, eidx, p['gat_W'], p['gat_asrc'],
                  p['gat_adst'], p['gat_b'])
    t = _graph_norm(t, p['gn_w'], p['gn_b'], p['gn_ms'])
    t = _leaky(t)
    return x + t


_PB = 4096


def _head_body(feat_ref, w1_ref, acc_ref):
    k = pl.program_id(0)

    @pl.when(k == 0)
    def _init():
        acc_ref[...] = jnp.zeros_like(acc_ref)

    acc_ref[...] += jnp.dot(feat_ref[...], w1_ref[...],
                            preferred_element_type=jnp.float32)


def _head_finish_body(acc_ref, b1_ref, w2_ref, b2_ref, out_ref):
    h = _leaky(acc_ref[...] + b1_ref[...])
    out_ref[...] = jnp.dot(h, w2_ref[...],
                           preferred_element_type=jnp.float32) + b2_ref[...]


def _mlp_head(feat, w1, b1, w2, b2):
    n = feat.shape[0]
    npad = ((n + _PB - 1) // _PB) * _PB
    featp = jnp.pad(feat, (0, npad - n)).reshape(1, npad)
    w1p = jnp.pad(w1, ((0, npad - n), (0, 0)))
    nk = npad // _PB
    acc = pl.pallas_call(
        _head_body,
        grid=(nk,),
        in_specs=[
            pl.BlockSpec((1, _PB), lambda k: (0, k)),
            pl.BlockSpec((_PB, 128), lambda k: (k, 0)),
        ],
        out_specs=pl.BlockSpec((1, 128), lambda k: (0, 0)),
        out_shape=jax.ShapeDtypeStruct((1, 128), jnp.float32),
    )(featp, w1p)
    out = pl.pallas_call(
        _head_finish_body,
        out_shape=jax.ShapeDtypeStruct((1, OUT_DIM), jnp.float32),
    )(acc, b1.reshape(1, 128), w2, b2.reshape(1, OUT_DIM))
    return out.reshape(OUT_DIM)


def kernel(x, edge_index, mask, weight, params):
    row, col = edge_index[0], edge_index[1]
    poi = jnp.where(mask, x, 0)[:, :HALF].reshape(P, -1).astype(jnp.int32)
    dist = jnp.where(mask, 0.0, x)[:, HALF:].reshape(P, -1)
    emb_poi = _sc_emb_lookup(params['emb'], poi)
    feat = jnp.concatenate([emb_poi, dist], axis=1)

    deg = jnp.ones((P,), jnp.float32).at[col].add(1.0)
    dinv = 1.0 / jnp.sqrt(deg)
    eidx = _prep_edges(row, col)

    feat = _leaky(_gcn_conv(feat, eidx, params['cov_in_W'],
                            params['cov_in_b'], dinv))
    for p in params['layers']:
        feat = _gcn_unit(feat, row, col, eidx, p, dinv)
    feat = _leaky(_gcn_conv_out(feat, eidx, params['cov_out_W'],
                                params['cov_out_b'], dinv))
    feat = feat.reshape(-1)
    return _mlp_head(feat, params['fc1_W'], params['fc1_b'],
                     params['fc2_W'], params['fc2_b'])
